# baseline (speedup 1.0000x reference)
import jax
import jax.numpy as jnp
from jax import lax
from jax.experimental import pallas as pl

TM = 512
KB = 2048
K = 8192
C = 32


def _vq_body(z_ref, zsq_ref, esq_ref, emb_ref, idx_ref, zq_ref):
    z = z_ref[...]
    zsq = zsq_ref[...]
    bestv = jnp.full((TM, 1), jnp.inf, dtype=jnp.float32)
    besti = jnp.zeros((TM, 1), dtype=jnp.int32)
    for kb in range(K // KB):
        emb = emb_ref[pl.ds(kb * KB, KB), :]
        q = lax.dot_general(z, emb, (((1,), (1,)), ((), ())),
                            preferred_element_type=jnp.float32)
        dist = (zsq - 2.0 * q) + esq_ref[0, pl.ds(kb * KB, KB)][None, :]
        mv = jnp.min(dist, axis=1, keepdims=True)
        gidx = lax.broadcasted_iota(jnp.int32, (TM, KB), 1) + kb * KB
        li = jnp.min(jnp.where(dist == mv, gidx, jnp.int32(2**30)),
                     axis=1, keepdims=True)
        upd = mv < bestv
        besti = jnp.where(upd, li, besti)
        bestv = jnp.where(upd, mv, bestv)
    idx_ref[...] = besti
    zq = jnp.zeros((TM, C), dtype=jnp.float32)
    for kb in range(K // KB):
        emb = emb_ref[pl.ds(kb * KB, KB), :]
        gidx = lax.broadcasted_iota(jnp.int32, (TM, KB), 1) + kb * KB
        oh = (gidx == besti).astype(jnp.float32)
        zq = zq + lax.dot_general(oh, emb, (((1,), (0,)), ((), ())),
                                  precision=lax.Precision.HIGHEST,
                                  preferred_element_type=jnp.float32)
    zq_ref[...] = zq


def kernel(feather, embedding):
    N, Cc, H, W = feather.shape
    z = jnp.transpose(feather, (0, 2, 3, 1)).reshape(-1, Cc)
    M = z.shape[0]
    zsq = jnp.sum(z * z, axis=1, keepdims=True)
    esq = jnp.sum(embedding * embedding, axis=1)[None, :]

    nearest_flat, zq_flat = pl.pallas_call(
        _vq_body,
        grid=(M // TM,),
        in_specs=[
            pl.BlockSpec((TM, Cc), lambda i: (i, 0)),
            pl.BlockSpec((TM, 1), lambda i: (i, 0)),
            pl.BlockSpec((1, K), lambda i: (0, 0)),
            pl.BlockSpec((K, Cc), lambda i: (0, 0)),
        ],
        out_specs=[
            pl.BlockSpec((TM, 1), lambda i: (i, 0)),
            pl.BlockSpec((TM, Cc), lambda i: (i, 0)),
        ],
        out_shape=[
            jax.ShapeDtypeStruct((M, 1), jnp.int32),
            jax.ShapeDtypeStruct((M, Cc), jnp.float32),
        ],
    )(z, zsq, esq, embedding)

    nearest = nearest_flat.reshape(N, H, W)
    zq = jnp.transpose(zq_flat.reshape(N, H, W, Cc), (0, 3, 1, 2))
    decoder_input = feather + lax.stop_gradient(zq - feather)
    return decoder_input, zq, nearest[:, None, :, :]


# speedup vs baseline: 2.1944x; 2.1944x over previous
import functools

import jax
import jax.numpy as jnp
from jax import lax
from jax.experimental import pallas as pl
from jax.experimental.pallas import tpu as pltpu
from jax.experimental.pallas import tpu_sc as plsc

TM = 128
K = 8192
C = 32

_NC, _NS = 2, 16
_NW = _NC * _NS
_M_TOTAL = 4608
_BPW = _M_TOTAL // _NW
_CHUNK = 72


def _sc_gather(table_hbm, idx_hbm, out_hbm, idx_v, rows_v, sem):
    wid = lax.axis_index("s") * _NC + lax.axis_index("c")
    base = wid * _BPW
    for j in range(_BPW // _CHUNK):
        pltpu.sync_copy(idx_hbm.at[pl.ds(base + j * _CHUNK, _CHUNK)],
                        idx_v.at[j])
        pltpu.async_copy(table_hbm.at[idx_v.at[j]],
                         rows_v.at[pl.ds(j * _CHUNK, _CHUNK)], sem).wait()
    pltpu.sync_copy(rows_v, out_hbm.at[pl.ds(base, _BPW)])


_sc_gather_call = functools.partial(
    pl.kernel,
    mesh=plsc.VectorSubcoreMesh(core_axis_name="c", subcore_axis_name="s"),
    out_type=jax.ShapeDtypeStruct((_M_TOTAL, C), jnp.float32),
    scratch_types=[
        pltpu.VMEM((_BPW // _CHUNK, _CHUNK), jnp.int32),
        pltpu.VMEM((_BPW, C), jnp.float32),
        pltpu.SemaphoreType.DMA,
    ],
    compiler_params=pltpu.CompilerParams(use_tc_tiling_on_sc=False),
)(_sc_gather)


def _vq_body(z_ref, zsq_ref, esq_ref, em2_ref, idx_ref):
    z = z_ref[...]
    zsq = zsq_ref[...]
    q = lax.dot_general(z, em2_ref[...], (((1,), (1,)), ((), ())),
                        preferred_element_type=jnp.float32)
    dist = (zsq + q) + esq_ref[...]
    mv = jnp.min(dist, axis=1, keepdims=True)
    gidx = lax.broadcasted_iota(jnp.int32, (TM, K), 1)
    idx = jnp.min(jnp.where(dist == mv, gidx, jnp.int32(K)), axis=1)
    idx_ref[...] = idx.reshape(1, 1, TM)


def kernel(feather, embedding):
    N, Cc, H, W = feather.shape
    z = jnp.transpose(feather, (0, 2, 3, 1)).reshape(-1, Cc)
    M = z.shape[0]
    zsq = jnp.sum(z * z, axis=1, keepdims=True)
    esq = jnp.sum(embedding * embedding, axis=1)[None, :]
    em2 = -2.0 * embedding

    nearest_blocks = pl.pallas_call(
        _vq_body,
        grid=(M // TM,),
        in_specs=[
            pl.BlockSpec((TM, Cc), lambda i: (i, 0)),
            pl.BlockSpec((TM, 1), lambda i: (i, 0)),
            pl.BlockSpec((1, K), lambda i: (0, 0)),
            pl.BlockSpec((K, Cc), lambda i: (0, 0)),
        ],
        out_specs=pl.BlockSpec((1, 1, TM), lambda i: (i, 0, 0)),
        out_shape=jax.ShapeDtypeStruct((M // TM, 1, TM), jnp.int32),
    )(z, zsq, esq, em2)

    nearest_flat = nearest_blocks.reshape(M)
    zq_flat = _sc_gather_call(embedding, nearest_flat)
    nearest = nearest_flat.reshape(N, H, W)
    zq = jnp.transpose(zq_flat.reshape(N, H, W, Cc), (0, 3, 1, 2))
    decoder_input = feather + lax.stop_gradient(zq - feather)
    return decoder_input, zq, nearest[:, None, :, :]


# speedup vs baseline: 2.3225x; 1.0584x over previous
import functools

import jax
import jax.numpy as jnp
from jax import lax
from jax.experimental import pallas as pl
from jax.experimental.pallas import tpu as pltpu
from jax.experimental.pallas import tpu_sc as plsc

TM = 256
K = 8192
C = 32

_NC, _NS = 2, 16
_NW = _NC * _NS
_M_TOTAL = 4608
_BPW = _M_TOTAL // _NW
_CHUNK = 72


def _sc_gather(table_hbm, idx_hbm, out_hbm, idx_v, rows_v, sem):
    wid = lax.axis_index("s") * _NC + lax.axis_index("c")
    base = wid * _BPW
    for j in range(_BPW // _CHUNK):
        pltpu.sync_copy(idx_hbm.at[pl.ds(base + j * _CHUNK, _CHUNK)],
                        idx_v.at[j])
        pltpu.async_copy(table_hbm.at[idx_v.at[j]],
                         rows_v.at[pl.ds(j * _CHUNK, _CHUNK)], sem).wait()
    pltpu.sync_copy(rows_v, out_hbm.at[pl.ds(base, _BPW)])


_sc_gather_call = functools.partial(
    pl.kernel,
    mesh=plsc.VectorSubcoreMesh(core_axis_name="c", subcore_axis_name="s"),
    out_type=jax.ShapeDtypeStruct((_M_TOTAL, C), jnp.float32),
    scratch_types=[
        pltpu.VMEM((_BPW // _CHUNK, _CHUNK), jnp.int32),
        pltpu.VMEM((_BPW, C), jnp.float32),
        pltpu.SemaphoreType.DMA,
    ],
    compiler_params=pltpu.CompilerParams(use_tc_tiling_on_sc=False),
)(_sc_gather)


def _vq_body(zm2_ref, zsq_ref, esq_ref, emb_ref, idx_ref):
    zm2 = zm2_ref[...]
    zsq = zsq_ref[...]
    q = lax.dot_general(zm2, emb_ref[...], (((1,), (1,)), ((), ())),
                        preferred_element_type=jnp.float32)
    dist = (zsq + q) + esq_ref[...]
    mv = jnp.min(dist, axis=1, keepdims=True)
    gidx = lax.broadcasted_iota(jnp.int32, (TM, K), 1)
    idx = jnp.min(jnp.where(dist == mv, gidx, jnp.int32(K)), axis=1)
    idx_ref[...] = idx.reshape(1, 1, TM)


def kernel(feather, embedding):
    N, Cc, H, W = feather.shape
    z = jnp.transpose(feather, (0, 2, 3, 1)).reshape(-1, Cc)
    M = z.shape[0]
    zsq = jnp.sum(z * z, axis=1, keepdims=True)
    esq = jnp.sum(embedding * embedding, axis=1)[None, :]
    zm2 = -2.0 * z

    nearest_blocks = pl.pallas_call(
        _vq_body,
        grid=(M // TM,),
        in_specs=[
            pl.BlockSpec((TM, Cc), lambda i: (i, 0)),
            pl.BlockSpec((TM, 1), lambda i: (i, 0)),
            pl.BlockSpec((1, K), lambda i: (0, 0)),
            pl.BlockSpec((K, Cc), lambda i: (0, 0)),
        ],
        out_specs=pl.BlockSpec((1, 1, TM), lambda i: (i, 0, 0)),
        out_shape=jax.ShapeDtypeStruct((M // TM, 1, TM), jnp.int32),
    )(zm2, zsq, esq, embedding)

    nearest_flat = nearest_blocks.reshape(M)
    zq_flat = _sc_gather_call(embedding, nearest_flat)
    nearest = nearest_flat.reshape(N, H, W)
    zq = jnp.transpose(zq_flat.reshape(N, H, W, Cc), (0, 3, 1, 2))
    decoder_input = feather + lax.stop_gradient(zq - feather)
    return decoder_input, zq, nearest[:, None, :, :]
